# baseline (speedup 1.0000x reference)
import functools

import jax
import jax.numpy as jnp
from jax import lax
from jax.experimental import pallas as pl
from jax.experimental.pallas import tpu as pltpu
from jax.experimental.pallas import tpu_sc as plsc

N = 10000
D = 128
H = 128
O = 128

NS = 16
G = 128
SB = 1280
CAP = SB + G
ACC_ROWS = 10240
DUMP = N
ZROWS = 32


def _sc_aggregate(x, src, dst, etype, upper):
    E = src.shape[0]
    assert E % SB == 0
    nsb = E // SB

    mesh = plsc.VectorSubcoreMesh(core_axis_name="c", subcore_axis_name="s")

    @functools.partial(
        pl.kernel,
        out_type=jax.ShapeDtypeStruct((4 * N, D), jnp.float32),
        mesh=mesh,
        scratch_types=[
            pltpu.VMEM_SHARED((ACC_ROWS, D), jnp.float32),
            pltpu.VMEM((SB,), jnp.int32),
            pltpu.VMEM((SB,), jnp.int32),
            pltpu.VMEM((SB,), jnp.int32),
            pltpu.VMEM((CAP // G, G), jnp.int32),
            pltpu.VMEM((CAP // G, G), jnp.int32),
            pltpu.VMEM((G, D), jnp.float32),
            pltpu.VMEM((ZROWS, D), jnp.float32),
            pltpu.SemaphoreType.DMA,
            pltpu.SemaphoreType.DMA,
        ],
        compiler_params=pltpu.CompilerParams(use_tc_tiling_on_sc=False,
                                             needs_layout_passes=False),
    )
    def body(x_hbm, src_hbm, dst_hbm, et_hbm, up_hbm, out_hbm,
             acc, gbuf, tbuf, dbuf, gcomp, scomp, rows, zbuf,
             semi, semg):
        core = lax.axis_index("c")
        tid = lax.axis_index("s")

        zv = jnp.zeros((16,), jnp.float32)

        def zrow(i, c):
            for j in range(D // 16):
                zbuf[i, pl.ds(j * 16, 16)] = zv
            return c

        lax.fori_loop(0, ZROWS, zrow, 0)

        dump_vec = jnp.full((16,), DUMP, dtype=jnp.int32)
        zero_ivec = jnp.zeros((16,), dtype=jnp.int32)
        iota = lax.iota(jnp.int32, 16)

        for t in range(4):
            g_hbm = src_hbm if t < 3 else up_hbm
            tmatch = t if t < 3 else 2

            @pl.when(core == (0 if t < 2 else 1))
            def _():
                zb = tid * (ACC_ROWS // NS)
                for k in range(ACC_ROWS // NS // ZROWS):
                    pltpu.sync_copy(zbuf, acc.at[pl.ds(zb + k * ZROWS, ZROWS)])
                plsc.subcore_barrier()

                lo = tid * nsb // NS
                hi = (tid + 1) * nsb // NS

                def sb_body(b, c):
                    s = b * SB
                    cp1 = pltpu.async_copy(g_hbm.at[pl.ds(s, SB)], gbuf, semi)
                    cp2 = pltpu.async_copy(et_hbm.at[pl.ds(s, SB)], tbuf, semi)
                    cp3 = pltpu.async_copy(dst_hbm.at[pl.ds(s, SB)], dbuf, semi)
                    cp1.wait()
                    cp2.wait()
                    cp3.wait()
                    def compact(i, cv):
                        tv = tbuf[pl.ds(i * 16, 16)]
                        dv = dbuf[pl.ds(i * 16, 16)]
                        gv = gbuf[pl.ds(i * 16, 16)]
                        m = tv == tmatch
                        pos = cv + iota
                        pr = lax.shift_right_logical(pos, 7)
                        pc = jnp.bitwise_and(pos, G - 1)
                        plsc.store_scatter(gcomp, [pr, pc], gv)
                        plsc.store_scatter(scomp, [pr, pc],
                                           jnp.where(m, dv, dump_vec))
                        return cv + 5 + jnp.bitwise_and(tv, 0)

                    cnt_v = lax.fori_loop(0, SB // 16, compact,
                                          jnp.zeros((16,), jnp.int32))
                    for j in range(G // 16):
                        pp = cnt_v + iota + j * 16
                        pr = lax.shift_right_logical(pp, 7)
                        pc = jnp.bitwise_and(pp, G - 1)
                        plsc.store_scatter(gcomp, [pr, pc], zero_ivec)
                        plsc.store_scatter(scomp, [pr, pc], dump_vec)
                    cnt = cnt_v[0]

                    def fire(k, c2):
                        pltpu.async_copy(x_hbm.at[gcomp.at[k]],
                                         rows, semg).wait()
                        pltpu.sync_copy(rows, acc.at[scomp.at[k]], add=True)
                        return c2

                    nfire = lax.shift_right_logical(cnt + (G - 1), 7)
                    lax.fori_loop(0, nfire, fire, 0)
                    return c

                lax.fori_loop(lo, hi, sb_body, 0)
                plsc.subcore_barrier()

                wb = tid * 624
                pltpu.sync_copy(acc.at[pl.ds(wb, 624)],
                                out_hbm.at[pl.ds(t * N + wb, 624)])

                @pl.when(tid == 0)
                def _():
                    pltpu.sync_copy(acc.at[pl.ds(16 * 624, N - 16 * 624)],
                                    out_hbm.at[pl.ds(t * N + 16 * 624,
                                                     N - 16 * 624)])

                plsc.subcore_barrier()

    return body(x, src, dst, etype, upper)


def _tc_dense(x, A, Wb, bb, Wr, br, Wm, bm, Wu, bu, Wo, bo):
    BN = 1000
    nblk = N // BN
    f32 = jnp.float32

    def body(x_ref, a_ref, wb_ref, bb_ref, wr_ref, br_ref, wm_ref, bm_ref,
             wu_ref, bu_ref, wo_ref, bo_ref, o_ref):
        xb = x_ref[...]
        a0 = a_ref[0]
        a1 = a_ref[1]
        a2 = a_ref[2]
        a3 = a_ref[3]
        bnd = jnp.maximum(
            jnp.dot(xb + a0, wb_ref[...], preferred_element_type=f32)
            + bb_ref[...], 0.0)
        rew = jnp.maximum(
            jnp.dot(xb + a1, wr_ref[...], preferred_element_type=f32)
            + br_ref[...], 0.0)
        u = (xb
             + jnp.dot(a2, wm_ref[0:D], preferred_element_type=f32)
             + jnp.dot(a3, wm_ref[D:2 * D], preferred_element_type=f32)
             + bm_ref[...])
        upp = jnp.maximum(
            jnp.dot(u, wu_ref[...], preferred_element_type=f32)
            + bu_ref[...], 0.0)
        o = jnp.maximum(
            jnp.dot(bnd, wo_ref[0:H], preferred_element_type=f32)
            + jnp.dot(rew, wo_ref[H:2 * H], preferred_element_type=f32)
            + jnp.dot(upp, wo_ref[2 * H:3 * H], preferred_element_type=f32)
            + bo_ref[...], 0.0)
        o_ref[...] = o

    full = lambda a: pl.BlockSpec(a.shape, lambda i: (0,) * a.ndim)
    return pl.pallas_call(
        body,
        grid=(nblk,),
        in_specs=[
            pl.BlockSpec((BN, D), lambda i: (i, 0)),
            pl.BlockSpec((4, BN, D), lambda i: (0, i, 0)),
            full(Wb), full(bb), full(Wr), full(br), full(Wm), full(bm),
            full(Wu), full(bu), full(Wo), full(bo),
        ],
        out_specs=pl.BlockSpec((BN, O), lambda i: (i, 0)),
        out_shape=jax.ShapeDtypeStruct((N, O), f32),
    )(x, A, Wb, bb, Wr, br, Wm, bm, Wu, bu, Wo, bo)


def kernel(x, edge_index, edge_type, upper_ind, cell_dimension,
           W_bnd, b_bnd, W_rew, b_rew, W_msg, b_msg, W_upd, b_upd,
           W_out, b_out):
    del cell_dimension
    src = edge_index[0]
    dst = edge_index[1]
    agg = _sc_aggregate(x, src, dst, edge_type, upper_ind)
    A = agg.reshape(4, N, D)
    return _tc_dense(x, A,
                     W_bnd, b_bnd.reshape(1, H),
                     W_rew, b_rew.reshape(1, H),
                     W_msg, b_msg.reshape(1, D),
                     W_upd, b_upd.reshape(1, H),
                     W_out, b_out.reshape(1, O))


# speedup vs baseline: 7.5387x; 7.5387x over previous
import functools

import jax
import jax.numpy as jnp
from jax import lax
from jax.experimental import pallas as pl
from jax.experimental.pallas import tpu as pltpu
from jax.experimental.pallas import tpu_sc as plsc

N = 10000
D = 128
H = 128
O = 128

NS = 16
HD = 64
G = 128
SB = 512
ACC_ROWS = 3 * N + 16
DUMP = 3 * N
ZR = ACC_ROWS // NS


def _sc_aggregate(xlo, xhi, src, dst, etype, upper, zrows):
    E = src.shape[0]
    assert E % SB == 0
    nsb = E // SB

    mesh = plsc.VectorSubcoreMesh(core_axis_name="c", subcore_axis_name="s")

    @functools.partial(
        pl.kernel,
        out_type=jax.ShapeDtypeStruct((4 * N, D), jnp.float32),
        mesh=mesh,
        scratch_types=[
            pltpu.VMEM_SHARED((ACC_ROWS, HD), jnp.float32),
            pltpu.VMEM((SB,), jnp.int32),
            pltpu.VMEM((SB,), jnp.int32),
            pltpu.VMEM((SB,), jnp.int32),
            pltpu.VMEM((SB // G, G), jnp.int32),
            pltpu.VMEM((G, HD), jnp.float32),
            pltpu.SemaphoreType.DMA,
            pltpu.SemaphoreType.DMA,
        ],
        compiler_params=pltpu.CompilerParams(use_tc_tiling_on_sc=False,
                                             needs_layout_passes=False),
    )
    def body(xlo_hbm, xhi_hbm, src_hbm, dst_hbm, et_hbm, up_hbm, z_hbm,
             out_hbm, acc, gbuf, tbuf, dbuf, sidx, rows, semi, semg):
        core = lax.axis_index("c")
        tid = lax.axis_index("s")
        lo = tid * nsb // NS
        hi = (tid + 1) * nsb // NS

        def sweep(x_half, g_arr, col, masked):
            def sb_body(b, c):
                s = b * SB
                cp1 = pltpu.async_copy(g_arr.at[pl.ds(s, SB)], gbuf, semi)
                cp2 = pltpu.async_copy(et_hbm.at[pl.ds(s, SB)], tbuf, semi)
                cp3 = pltpu.async_copy(dst_hbm.at[pl.ds(s, SB)], dbuf, semi)
                cp1.wait()
                cp2.wait()
                cp3.wait()
                for i in range(SB // 16):
                    tv = tbuf[pl.ds(i * 16, 16)]
                    dv = dbuf[pl.ds(i * 16, 16)]
                    if masked:
                        sv = jnp.where(tv == 2, dv, N)
                    else:
                        sv = dv + tv * N
                    sidx[i // 8, pl.ds((i % 8) * 16, 16)] = sv
                for k in range(SB // G):
                    pltpu.async_copy(
                        x_half.at[gbuf.at[pl.ds(k * G, G)]], rows,
                        semg).wait()
                    pltpu.sync_copy(rows, acc.at[sidx.at[k]], add=True)
                return c

            lax.fori_loop(lo, hi, sb_body, 0)
            plsc.subcore_barrier()

        for c in range(2):
            x_half = xlo_hbm if c == 0 else xhi_hbm
            col = c * HD

            @pl.when(core == c)
            def _():
                pltpu.sync_copy(z_hbm, acc.at[pl.ds(tid * ZR, ZR)])
                plsc.subcore_barrier()
                sweep(x_half, src_hbm, col, masked=False)
                wb = tid * 1872
                pltpu.sync_copy(
                    acc.at[pl.ds(wb, 1872)],
                    out_hbm.at[pl.ds(wb, 1872), pl.ds(col, HD)])

                @pl.when(tid == 0)
                def _():
                    pltpu.sync_copy(
                        acc.at[pl.ds(16 * 1872, 3 * N - 16 * 1872)],
                        out_hbm.at[pl.ds(16 * 1872, 3 * N - 16 * 1872),
                                   pl.ds(col, HD)])

                plsc.subcore_barrier()

                pltpu.sync_copy(z_hbm.at[pl.ds(0, 626)],
                                acc.at[pl.ds(tid * 626, 626)])
                plsc.subcore_barrier()
                sweep(x_half, up_hbm, col, masked=True)
                wb2 = tid * 624
                pltpu.sync_copy(
                    acc.at[pl.ds(wb2, 624)],
                    out_hbm.at[pl.ds(3 * N + wb2, 624), pl.ds(col, HD)])

                @pl.when(tid == 0)
                def _():
                    pltpu.sync_copy(
                        acc.at[pl.ds(16 * 624, N - 16 * 624)],
                        out_hbm.at[pl.ds(3 * N + 16 * 624, N - 16 * 624),
                                   pl.ds(col, HD)])

                plsc.subcore_barrier()

    return body(xlo, xhi, src, dst, etype, upper, zrows)


def _tc_dense(x, A, Wb, bb, Wr, br, Wm, bm, Wu, bu, Wo, bo):
    BN = 1000
    nblk = N // BN
    f32 = jnp.float32

    def body(x_ref, a_ref, wb_ref, bb_ref, wr_ref, br_ref, wm_ref, bm_ref,
             wu_ref, bu_ref, wo_ref, bo_ref, o_ref):
        xb = x_ref[...]
        a0 = a_ref[0]
        a1 = a_ref[1]
        a2 = a_ref[2]
        a3 = a_ref[3]
        bnd = jnp.maximum(
            jnp.dot(xb + a0, wb_ref[...], preferred_element_type=f32)
            + bb_ref[...], 0.0)
        rew = jnp.maximum(
            jnp.dot(xb + a1, wr_ref[...], preferred_element_type=f32)
            + br_ref[...], 0.0)
        u = (xb
             + jnp.dot(a2, wm_ref[0:D], preferred_element_type=f32)
             + jnp.dot(a3, wm_ref[D:2 * D], preferred_element_type=f32)
             + bm_ref[...])
        upp = jnp.maximum(
            jnp.dot(u, wu_ref[...], preferred_element_type=f32)
            + bu_ref[...], 0.0)
        o = jnp.maximum(
            jnp.dot(bnd, wo_ref[0:H], preferred_element_type=f32)
            + jnp.dot(rew, wo_ref[H:2 * H], preferred_element_type=f32)
            + jnp.dot(upp, wo_ref[2 * H:3 * H], preferred_element_type=f32)
            + bo_ref[...], 0.0)
        o_ref[...] = o

    full = lambda a: pl.BlockSpec(a.shape, lambda i: (0,) * a.ndim)
    return pl.pallas_call(
        body,
        grid=(nblk,),
        in_specs=[
            pl.BlockSpec((BN, D), lambda i: (i, 0)),
            pl.BlockSpec((4, BN, D), lambda i: (0, i, 0)),
            full(Wb), full(bb), full(Wr), full(br), full(Wm), full(bm),
            full(Wu), full(bu), full(Wo), full(bo),
        ],
        out_specs=pl.BlockSpec((BN, O), lambda i: (i, 0)),
        out_shape=jax.ShapeDtypeStruct((N, O), f32),
    )(x, A, Wb, bb, Wr, br, Wm, bm, Wu, bu, Wo, bo)


def kernel(x, edge_index, edge_type, upper_ind, cell_dimension,
           W_bnd, b_bnd, W_rew, b_rew, W_msg, b_msg, W_upd, b_upd,
           W_out, b_out):
    del cell_dimension
    src = edge_index[0]
    dst = edge_index[1]
    xlo = x[:, :HD]
    xhi = x[:, HD:]
    zrows = jnp.zeros((ZR, HD), jnp.float32)
    agg = _sc_aggregate(xlo, xhi, src, dst, edge_type, upper_ind, zrows)
    A = agg.reshape(4, N, D)
    return _tc_dense(x, A,
                     W_bnd, b_bnd.reshape(1, H),
                     W_rew, b_rew.reshape(1, H),
                     W_msg, b_msg.reshape(1, D),
                     W_upd, b_upd.reshape(1, H),
                     W_out, b_out.reshape(1, O))
